# baseline (speedup 1.0000x reference)
import functools

import jax
import jax.numpy as jnp
from jax import lax
from jax.experimental import pallas as pl
from jax.experimental.pallas import tpu as pltpu
from jax.experimental.pallas import tpu_sc as plsc

_N = 1024 * 1024
_NC = 2
_NS = 16
_NW = _NC * _NS
_CHUNK = _N // _NW
_NB = 8
_SUB = _CHUNK // _NB
_L = 16


def _sc_body(w_hbm, n_hbm, o_hbm,
             w0, w1, n0, n1, o0, o1,
             sw0, sw1, sn0, sn1, so0, so1):
    wid = lax.axis_index("s") * _NC + lax.axis_index("c")
    base = wid * _CHUNK
    wv = (w0, w1)
    nv = (n0, n1)
    ov = (o0, o1)
    sw = (sw0, sw1)
    sn = (sn0, sn1)
    so = (so0, so1)

    def start_in(g):
        b = g & 1
        off = base + g * _SUB
        hw = pltpu.async_copy(w_hbm.at[pl.ds(off, _SUB)], wv[b], sw[b])
        hn = pltpu.async_copy(n_hbm.at[pl.ds(off, _SUB)], nv[b], sn[b])
        return hw, hn

    in_h = [None, None]
    out_h = [None, None]
    in_h[0] = start_in(0)
    for g in range(_NB):
        b = g & 1
        if g + 1 < _NB:
            in_h[1 - b] = start_in(g + 1)
        hw, hn = in_h[b]
        hw.wait()
        hn.wait()
        if out_h[b] is not None:
            out_h[b].wait()
        wb, nb, ob = wv[b], nv[b], ov[b]

        @plsc.parallel_loop(0, _SUB, step=_L, unroll=16)
        def _loop(i):
            x = (nb[pl.ds(i, _L)] - wb[pl.ds(i, _L)]) * 10.0
            ob[pl.ds(i, _L)] = 1.0 / (1.0 + jnp.exp(x))

        out_h[b] = pltpu.async_copy(
            ov[b], o_hbm.at[pl.ds(base + g * _SUB, _SUB)], so[b])
    out_h[0].wait()
    out_h[1].wait()


_sc_kernel = functools.partial(
    pl.kernel,
    mesh=plsc.VectorSubcoreMesh(core_axis_name="c", subcore_axis_name="s"),
    out_type=jax.ShapeDtypeStruct((_N,), jnp.float32),
    scratch_types=(
        [pltpu.VMEM((_SUB,), jnp.float32) for _ in range(6)]
        + [pltpu.SemaphoreType.DMA for _ in range(6)]
    ),
)(_sc_body)


def kernel(weights, noises):
    return _sc_kernel(weights, noises)


# speedup vs baseline: 3.4229x; 3.4229x over previous
import jax
import jax.numpy as jnp
from jax.experimental import pallas as pl

_N = 1024 * 1024
_STEPS = 8
_BLOCK = _N // _STEPS


def _gen_kernel(w_ref, n_ref, o_ref):
    o_ref[...] = jax.nn.sigmoid((w_ref[...] - n_ref[...]) * 10.0)


def kernel(weights, noises):
    return pl.pallas_call(
        _gen_kernel,
        out_shape=jax.ShapeDtypeStruct((_N,), jnp.float32),
        grid=(_STEPS,),
        in_specs=[
            pl.BlockSpec((_BLOCK,), lambda i: (i,)),
            pl.BlockSpec((_BLOCK,), lambda i: (i,)),
        ],
        out_specs=pl.BlockSpec((_BLOCK,), lambda i: (i,)),
    )(weights, noises)


# speedup vs baseline: 5.3654x; 1.5675x over previous
import jax
import jax.numpy as jnp
from jax.experimental import pallas as pl

_N = 1024 * 1024
_STEPS = 2
_BLOCK = _N // _STEPS


def _gen_kernel(w_ref, n_ref, o_ref):
    o_ref[...] = jax.nn.sigmoid((w_ref[...] - n_ref[...]) * 10.0)


def kernel(weights, noises):
    return pl.pallas_call(
        _gen_kernel,
        out_shape=jax.ShapeDtypeStruct((_N,), jnp.float32),
        grid=(_STEPS,),
        in_specs=[
            pl.BlockSpec((_BLOCK,), lambda i: (i,)),
            pl.BlockSpec((_BLOCK,), lambda i: (i,)),
        ],
        out_specs=pl.BlockSpec((_BLOCK,), lambda i: (i,)),
    )(weights, noises)
